# baseline (speedup 1.0000x reference)
import functools

import jax
import jax.numpy as jnp
from jax import lax
from jax.experimental import pallas as pl
from jax.experimental.pallas import tpu as pltpu
from jax.experimental.pallas import tpu_sc as plsc

B, L, V, D = 4096, 200, 8192, 64
BL = B * L
NC, NS = 2, 16
NW = NC * NS
ROWS_W = B // NW
POS_W = ROWS_W * L
NVEC = POS_W // 16
SPLIT = 128
REST = L - SPLIT
NB = 4
NOUTER = ROWS_W // NB


@functools.partial(
    pl.kernel,
    mesh=plsc.VectorSubcoreMesh(core_axis_name="c", subcore_axis_name="s"),
    out_type=jax.ShapeDtypeStruct((B, L, D), jnp.float32),
    scratch_types=[
        pltpu.VMEM((POS_W,), jnp.int32),
        pltpu.VMEM((ROWS_W,), jnp.int32),
        pltpu.VMEM((NB, L, D), jnp.float32),
        pltpu.SemaphoreType.DMA((NB,)),
        pltpu.SemaphoreType.DMA((NB,)),
    ],
    compiler_params=pltpu.CompilerParams(
        needs_layout_passes=False, use_tc_tiling_on_sc=False
    ),
)
def _seg_embed(table_hbm, tok_hbm, sep_hbm, out_hbm, idx_v, sep_v, rows_v,
               gsem, wsem):
    wid = lax.axis_index("s") * NC + lax.axis_index("c")
    pltpu.sync_copy(tok_hbm.at[pl.ds(wid * POS_W, POS_W)], idx_v)
    pltpu.sync_copy(sep_hbm.at[pl.ds(wid * ROWS_W, ROWS_W)], sep_v)

    lanes = lax.iota(jnp.int32, 16)

    def vec_body(g, carry):
        p = g * 16 + lanes
        l_pos = lax.rem(p, L)
        row = lax.div(p, L)
        sep16 = plsc.load_gather(sep_v, [row])
        tok = idx_v[pl.ds(g * 16, 16)]
        idx_v[pl.ds(g * 16, 16)] = jnp.where(l_pos > sep16, tok + V, tok)
        return carry

    def gathers(r, b):
        off = r * L
        c1 = pltpu.make_async_copy(
            table_hbm.at[idx_v.at[pl.ds(off, SPLIT)]],
            rows_v.at[b, pl.ds(0, SPLIT)], gsem.at[b])
        c2 = pltpu.make_async_copy(
            table_hbm.at[idx_v.at[pl.ds(off + SPLIT, REST)]],
            rows_v.at[b, pl.ds(SPLIT, REST)], gsem.at[b])
        return c1, c2

    def store(r, b):
        return pltpu.make_async_copy(
            rows_v.at[b], out_hbm.at[wid * ROWS_W + r], wsem.at[b])

    lax.fori_loop(0, NVEC, vec_body, 0)

    def ring(ro, carry):
        for b in range(NB):
            r = ro * NB + b
            @pl.when(ro > 0)
            def _():
                store(r - NB, b).wait()
            c1, c2 = gathers(r, b)
            c1.start()
            c2.start()
        for b in range(NB):
            r = ro * NB + b
            c1, c2 = gathers(r, b)
            c1.wait()
            c2.wait()
            store(r, b).start()
        return carry

    lax.fori_loop(0, NOUTER, ring, 0)
    for b in range(NB):
        store((NOUTER - 1) * NB + b, b).wait()


def kernel(inputs, sep_token_indices, seg_emb1, seg_emb2):
    table = jnp.concatenate([seg_emb1, seg_emb2], axis=0)
    tok = inputs.astype(jnp.int32).reshape(BL)
    sep = sep_token_indices.astype(jnp.int32)
    return _seg_embed(table, tok, sep)
